# baseline (speedup 1.0000x reference)
import functools

import jax
import jax.numpy as jnp
from jax import lax
from jax.experimental import pallas as pl
from jax.experimental.pallas import tpu as pltpu
from jax.experimental.pallas import tpu_sc as plsc

_NC = 2
_NS = 16
_NW = _NC * _NS
_LANES = 16

_CHUNK = 32
_NROW = 3


def _make_sc_embed(n_batch, seq_len, hidden):
    s_per_w = seq_len // _NW
    halves = s_per_w // _CHUNK
    n_chunks = n_batch * halves
    hgroups = hidden // _LANES
    n_tokens = n_batch * seq_len

    mesh = plsc.VectorSubcoreMesh(core_axis_name="c", subcore_axis_name="s")

    @functools.partial(
        pl.kernel,
        mesh=mesh,
        out_type=jax.ShapeDtypeStruct((n_tokens, hidden), jnp.float32),
        scratch_types=(
            [pltpu.VMEM((n_batch * s_per_w,), jnp.int32),
             pltpu.VMEM((hidden,), jnp.float32),
             pltpu.VMEM((s_per_w, hidden), jnp.float32)]
            + [pltpu.VMEM((_CHUNK, hidden), jnp.float32) for _ in range(_NROW)]
            + [pltpu.SemaphoreType.DMA for _ in range(2 * _NROW + 1)]
        ),
    )
    def sc_embed(ids_hbm, wtab_hbm, ptab_hbm, ttab_hbm, out_hbm,
                 idx_v, type_v, pos_v, *bufs_and_sems):
        rows = bufs_and_sems[:_NROW]
        gsem = bufs_and_sems[_NROW:2 * _NROW]
        osem = bufs_and_sems[2 * _NROW:3 * _NROW]
        psem = bufs_and_sems[3 * _NROW]

        wid = lax.axis_index("s") * _NC + lax.axis_index("c")
        s_base = wid * s_per_w
        for b in range(n_batch):
            pltpu.sync_copy(ids_hbm.at[pl.ds(b * seq_len + s_base, s_per_w)],
                            idx_v.at[pl.ds(b * s_per_w, s_per_w)])
        pos_cp = pltpu.async_copy(ptab_hbm.at[pl.ds(s_base, s_per_w)],
                                  pos_v, psem)
        pltpu.sync_copy(ttab_hbm.at[0], type_v)

        def issue_gather(c):
            return pltpu.async_copy(
                wtab_hbm.at[idx_v.at[pl.ds(c * _CHUNK, _CHUNK)]],
                rows[c % _NROW], gsem[c % _NROW])

        def issue_store(c):
            b, half = divmod(c, halves)
            off = b * seq_len + s_base + half * _CHUNK
            return pltpu.async_copy(
                rows[c % _NROW], out_hbm.at[pl.ds(off, _CHUNK)],
                osem[c % _NROW])

        g = [None] * n_chunks
        o = [None] * n_chunks
        for c in range(min(_NROW, n_chunks)):
            g[c] = issue_gather(c)

        pos_cp.wait()

        @plsc.parallel_loop(0, s_per_w, unroll=1)
        def pre_add(r):
            for h in range(hgroups):
                sl = pl.ds(h * _LANES, _LANES)
                plsc.addupdate(pos_v.at[r, sl], type_v[sl])

        for c in range(n_chunks):
            if c >= 2:
                o[c - 2].wait()
                if c + 1 < n_chunks:
                    g[c + 1] = issue_gather(c + 1)
            g[c].wait()
            rbuf = rows[c % _NROW]
            pbase = (c % halves) * _CHUNK

            @plsc.parallel_loop(0, _CHUNK, unroll=2)
            def row_body(r):
                for h in range(hgroups):
                    sl = pl.ds(h * _LANES, _LANES)
                    plsc.addupdate(rbuf.at[r, sl], pos_v[pbase + r, sl])

            o[c] = issue_store(c)

        for c in range(max(0, n_chunks - 2), n_chunks):
            o[c].wait()

    return sc_embed


def kernel(input_ids, word_emb, pos_emb, type_emb):
    b, s = input_ids.shape
    hidden = word_emb.shape[1]
    assert s % (_NW * _CHUNK) == 0

    ids_flat = input_ids.reshape(-1).astype(jnp.int32)
    fn = _make_sc_embed(b, s, hidden)
    out = fn(ids_flat, word_emb, pos_emb, type_emb)
    return out.reshape(b, s, hidden)


# speedup vs baseline: 1.0047x; 1.0047x over previous
import functools

import jax
import jax.numpy as jnp
from jax import lax
from jax.experimental import pallas as pl
from jax.experimental.pallas import tpu as pltpu
from jax.experimental.pallas import tpu_sc as plsc

_NC = 2
_NS = 16
_NW = _NC * _NS
_LANES = 16

_CHUNK = 16
_NROW = 6
_LEAD = 4


def _make_sc_embed(n_batch, seq_len, hidden):
    s_per_w = seq_len // _NW
    halves = s_per_w // _CHUNK
    n_chunks = n_batch * halves
    hgroups = hidden // _LANES
    n_tokens = n_batch * seq_len

    mesh = plsc.VectorSubcoreMesh(core_axis_name="c", subcore_axis_name="s")

    @functools.partial(
        pl.kernel,
        mesh=mesh,
        out_type=jax.ShapeDtypeStruct((n_tokens, hidden), jnp.float32),
        scratch_types=(
            [pltpu.VMEM((n_batch * s_per_w,), jnp.int32),
             pltpu.VMEM((hidden,), jnp.float32),
             pltpu.VMEM((s_per_w, hidden), jnp.float32)]
            + [pltpu.VMEM((_CHUNK, hidden), jnp.float32) for _ in range(_NROW)]
            + [pltpu.SemaphoreType.DMA for _ in range(2 * _NROW + 1)]
        ),
    )
    def sc_embed(ids_hbm, wtab_hbm, ptab_hbm, ttab_hbm, out_hbm,
                 idx_v, type_v, pos_v, *bufs_and_sems):
        rows = bufs_and_sems[:_NROW]
        gsem = bufs_and_sems[_NROW:2 * _NROW]
        osem = bufs_and_sems[2 * _NROW:3 * _NROW]
        psem = bufs_and_sems[3 * _NROW]

        wid = lax.axis_index("s") * _NC + lax.axis_index("c")
        s_base = wid * s_per_w
        for b in range(n_batch):
            pltpu.sync_copy(ids_hbm.at[pl.ds(b * seq_len + s_base, s_per_w)],
                            idx_v.at[pl.ds(b * s_per_w, s_per_w)])
        pos_cp = pltpu.async_copy(ptab_hbm.at[pl.ds(s_base, s_per_w)],
                                  pos_v, psem)
        pltpu.sync_copy(ttab_hbm.at[0], type_v)

        def issue_gather(c):
            return pltpu.async_copy(
                wtab_hbm.at[idx_v.at[pl.ds(c * _CHUNK, _CHUNK)]],
                rows[c % _NROW], gsem[c % _NROW])

        def issue_store(c):
            b, part = divmod(c, halves)
            off = b * seq_len + s_base + part * _CHUNK
            return pltpu.async_copy(
                rows[c % _NROW], out_hbm.at[pl.ds(off, _CHUNK)],
                osem[c % _NROW])

        g = [None] * n_chunks
        o = [None] * n_chunks
        for c in range(min(_LEAD, n_chunks)):
            g[c] = issue_gather(c)

        pos_cp.wait()

        @plsc.parallel_loop(0, s_per_w, unroll=1)
        def pre_add(r):
            for h in range(hgroups):
                sl = pl.ds(h * _LANES, _LANES)
                plsc.addupdate(pos_v.at[r, sl], type_v[sl])

        for c in range(n_chunks):
            nc = c + _LEAD
            if nc < n_chunks:
                if nc >= _NROW:
                    o[nc - _NROW].wait()
                g[nc] = issue_gather(nc)
            g[c].wait()
            rbuf = rows[c % _NROW]
            pbase = (c % halves) * _CHUNK

            @plsc.parallel_loop(0, _CHUNK, unroll=1)
            def row_body(r):
                for h in range(hgroups):
                    sl = pl.ds(h * _LANES, _LANES)
                    plsc.addupdate(rbuf.at[r, sl], pos_v[pbase + r, sl])

            o[c] = issue_store(c)

        for c in range(max(0, n_chunks - _NROW), n_chunks):
            o[c].wait()

    return sc_embed


def kernel(input_ids, word_emb, pos_emb, type_emb):
    b, s = input_ids.shape
    hidden = word_emb.shape[1]
    assert s % (_NW * _CHUNK) == 0

    ids_flat = input_ids.reshape(-1).astype(jnp.int32)
    fn = _make_sc_embed(b, s, hidden)
    out = fn(ids_flat, word_emb, pos_emb, type_emb)
    return out.reshape(b, s, hidden)


# speedup vs baseline: 1.0710x; 1.0660x over previous
import functools

import jax
import jax.numpy as jnp
from jax import lax
from jax.experimental import pallas as pl
from jax.experimental.pallas import tpu as pltpu
from jax.experimental.pallas import tpu_sc as plsc

_NC = 2
_NS = 16
_NW = _NC * _NS
_LANES = 16

_CHUNK = 32
_NROW = 3
_LEAD = 1


def _make_sc_embed(n_batch, seq_len, hidden):
    s_per_w = seq_len // _NW
    halves = s_per_w // _CHUNK
    n_chunks = n_batch * halves
    hgroups = hidden // _LANES
    n_tokens = n_batch * seq_len

    mesh = plsc.VectorSubcoreMesh(core_axis_name="c", subcore_axis_name="s")

    @functools.partial(
        pl.kernel,
        mesh=mesh,
        out_type=jax.ShapeDtypeStruct((n_tokens, hidden), jnp.float32),
        scratch_types=(
            [pltpu.VMEM((n_batch * s_per_w,), jnp.int32),
             pltpu.VMEM((hidden,), jnp.float32),
             pltpu.VMEM((s_per_w, hidden), jnp.float32)]
            + [pltpu.VMEM((_CHUNK, hidden), jnp.float32) for _ in range(_NROW)]
            + [pltpu.SemaphoreType.DMA for _ in range(2 * _NROW + 2)]
        ),
    )
    def sc_embed(ids_hbm, wtab_hbm, ptab_hbm, ttab_hbm, out_hbm,
                 idx_v, type_v, pos_v, *bufs_and_sems):
        rows = bufs_and_sems[:_NROW]
        gsem = bufs_and_sems[_NROW:2 * _NROW]
        osem = bufs_and_sems[2 * _NROW:3 * _NROW]
        psem = bufs_and_sems[3 * _NROW]
        isem = bufs_and_sems[3 * _NROW + 1]

        wid = lax.axis_index("s") * _NC + lax.axis_index("c")
        s_base = wid * s_per_w
        pos_cp = pltpu.async_copy(ptab_hbm.at[pl.ds(s_base, s_per_w)],
                                  pos_v, psem)
        id_cps = [
            pltpu.async_copy(ids_hbm.at[pl.ds(b * seq_len + s_base, s_per_w)],
                             idx_v.at[pl.ds(b * s_per_w, s_per_w)], isem)
            for b in range(n_batch)
        ]
        type_cp = pltpu.async_copy(ttab_hbm.at[0], type_v, isem)
        for cp in id_cps:
            cp.wait()

        def issue_gather(c):
            return pltpu.async_copy(
                wtab_hbm.at[idx_v.at[pl.ds(c * _CHUNK, _CHUNK)]],
                rows[c % _NROW], gsem[c % _NROW])

        def issue_store(c):
            b, part = divmod(c, halves)
            off = b * seq_len + s_base + part * _CHUNK
            return pltpu.async_copy(
                rows[c % _NROW], out_hbm.at[pl.ds(off, _CHUNK)],
                osem[c % _NROW])

        g = [None] * n_chunks
        o = [None] * n_chunks
        for c in range(min(_LEAD, n_chunks)):
            g[c] = issue_gather(c)

        type_cp.wait()
        pos_cp.wait()

        @plsc.parallel_loop(0, s_per_w, unroll=1)
        def pre_add(r):
            for h in range(hgroups):
                sl = pl.ds(h * _LANES, _LANES)
                plsc.addupdate(pos_v.at[r, sl], type_v[sl])

        for c in range(n_chunks):
            nc = c + _LEAD
            if nc < n_chunks:
                if nc >= _NROW:
                    o[nc - _NROW].wait()
                g[nc] = issue_gather(nc)
            g[c].wait()
            rbuf = rows[c % _NROW]
            pbase = (c % halves) * _CHUNK

            @plsc.parallel_loop(0, _CHUNK, unroll=1)
            def row_body(r):
                for h in range(hgroups):
                    sl = pl.ds(h * _LANES, _LANES)
                    plsc.addupdate(rbuf.at[r, sl], pos_v[pbase + r, sl])

            o[c] = issue_store(c)

        for c in range(max(0, n_chunks - _NROW), n_chunks):
            o[c].wait()

    return sc_embed


def kernel(input_ids, word_emb, pos_emb, type_emb):
    b, s = input_ids.shape
    hidden = word_emb.shape[1]
    assert s % (_NW * _CHUNK) == 0

    ids_flat = input_ids.reshape(-1).astype(jnp.int32)
    fn = _make_sc_embed(b, s, hidden)
    out = fn(ids_flat, word_emb, pos_emb, type_emb)
    return out.reshape(b, s, hidden)


# speedup vs baseline: 1.0914x; 1.0191x over previous
import functools

import jax
import jax.numpy as jnp
from jax import lax
from jax.experimental import pallas as pl
from jax.experimental.pallas import tpu as pltpu
from jax.experimental.pallas import tpu_sc as plsc

_NC = 2
_NS = 16
_NW = _NC * _NS
_LANES = 16

_CHUNK = 32
_NROW = 3
_LEAD = 1


def _make_sc_embed(n_batch, seq_len, hidden):
    s_per_w = seq_len // _NW
    halves = s_per_w // _CHUNK
    n_chunks = n_batch * halves
    hgroups = hidden // _LANES
    n_tokens = n_batch * seq_len

    mesh = plsc.VectorSubcoreMesh(core_axis_name="c", subcore_axis_name="s")

    @functools.partial(
        pl.kernel,
        mesh=mesh,
        out_type=jax.ShapeDtypeStruct((n_tokens, hidden), jnp.float32),
        scratch_types=(
            [pltpu.VMEM((n_batch * s_per_w,), jnp.int32),
             pltpu.VMEM((hidden,), jnp.float32),
             pltpu.VMEM((s_per_w, hidden), jnp.float32)]
            + [pltpu.VMEM((_CHUNK, hidden), jnp.float32) for _ in range(_NROW)]
            + [pltpu.SemaphoreType.DMA for _ in range(2 * _NROW + 2)]
        ),
    )
    def sc_embed(ids_hbm, wtab_hbm, ptab_hbm, ttab_hbm, out_hbm,
                 idx_v, type_v, pos_v, *bufs_and_sems):
        rows = bufs_and_sems[:_NROW]
        gsem = bufs_and_sems[_NROW:2 * _NROW]
        osem = bufs_and_sems[2 * _NROW:3 * _NROW]
        psem = bufs_and_sems[3 * _NROW]
        isem = bufs_and_sems[3 * _NROW + 1]

        wid = lax.axis_index("s") * _NC + lax.axis_index("c")
        s_base = wid * s_per_w
        half_rows = s_per_w // 2
        pos_cps = [
            pltpu.async_copy(
                ptab_hbm.at[pl.ds(s_base + i * half_rows, half_rows)],
                pos_v.at[pl.ds(i * half_rows, half_rows)], psem)
            for i in range(2)
        ]
        id_cps = [
            pltpu.async_copy(ids_hbm.at[pl.ds(b * seq_len + s_base, s_per_w)],
                             idx_v.at[pl.ds(b * s_per_w, s_per_w)], isem)
            for b in range(n_batch)
        ]
        type_cp = pltpu.async_copy(ttab_hbm.at[0], type_v, isem)
        for cp in id_cps:
            cp.wait()

        def issue_gather(c):
            return pltpu.async_copy(
                wtab_hbm.at[idx_v.at[pl.ds(c * _CHUNK, _CHUNK)]],
                rows[c % _NROW], gsem[c % _NROW])

        def issue_store(c):
            b, part = divmod(c, halves)
            off = b * seq_len + s_base + part * _CHUNK
            return pltpu.async_copy(
                rows[c % _NROW], out_hbm.at[pl.ds(off, _CHUNK)],
                osem[c % _NROW])

        g = [None] * n_chunks
        o = [None] * n_chunks
        for c in range(min(_LEAD, n_chunks)):
            g[c] = issue_gather(c)

        def pre_add_half(lo):
            @plsc.parallel_loop(lo, lo + half_rows, unroll=1)
            def pre_add(r):
                for h in range(hgroups):
                    sl = pl.ds(h * _LANES, _LANES)
                    plsc.addupdate(pos_v.at[r, sl], type_v[sl])

        type_cp.wait()
        pos_cps[0].wait()
        pre_add_half(0)

        for c in range(n_chunks):
            nc = c + _LEAD
            if nc < n_chunks:
                if nc >= _NROW:
                    o[nc - _NROW].wait()
                g[nc] = issue_gather(nc)
            g[c].wait()
            rbuf = rows[c % _NROW]
            pbase = (c % halves) * _CHUNK

            @plsc.parallel_loop(0, _CHUNK, unroll=1)
            def row_body(r):
                for h in range(hgroups):
                    sl = pl.ds(h * _LANES, _LANES)
                    plsc.addupdate(rbuf.at[r, sl], pos_v[pbase + r, sl])

            o[c] = issue_store(c)
            if c == 0:
                pos_cps[1].wait()
                pre_add_half(half_rows)

        for c in range(max(0, n_chunks - _NROW), n_chunks):
            o[c].wait()

    return sc_embed


def kernel(input_ids, word_emb, pos_emb, type_emb):
    b, s = input_ids.shape
    hidden = word_emb.shape[1]
    assert s % (_NW * _CHUNK) == 0

    ids_flat = input_ids.reshape(-1).astype(jnp.int32)
    fn = _make_sc_embed(b, s, hidden)
    out = fn(ids_flat, word_emb, pos_emb, type_emb)
    return out.reshape(b, s, hidden)
